# baseline (speedup 1.0000x reference)
import jax
import jax.numpy as jnp
from jax.experimental import pallas as pl
from jax.experimental.pallas import tpu as pltpu

_K = 1024
_B, _C, _N = 4, 6, 16384
_TILE = 2048


def _scores_body(pts_ref, w1_ref, b1_ref, w2_ref, b2_ref, w3_ref, b3_ref,
                 wlw_ref, wlb_ref, out_ref):
    acc = None
    for b in range(_B):
        x = pts_ref[b]
        h = jnp.dot(w1_ref[...], x, preferred_element_type=jnp.float32)
        h = jnp.maximum(h + b1_ref[...], 0.0)
        h = jnp.dot(w2_ref[...], h, preferred_element_type=jnp.float32)
        h = jnp.maximum(h + b2_ref[...], 0.0)
        h = jnp.dot(w3_ref[...], h, preferred_element_type=jnp.float32)
        h = h + b3_ref[...]
        s = jnp.dot(wlw_ref[...], h, preferred_element_type=jnp.float32)
        s = s + wlb_ref[...]
        acc = s if acc is None else acc + s
    out_ref[...] = acc * 0.25


def _mean_scores(src_pts, W1, b1, W2, b2, W3, b3, wl_w, wl_b):
    grid = (_N // _TILE,)
    out = pl.pallas_call(
        _scores_body,
        grid=grid,
        in_specs=[
            pl.BlockSpec((_B, _C, _TILE), lambda i: (0, 0, i)),
            pl.BlockSpec((64, _C), lambda i: (0, 0)),
            pl.BlockSpec((64, 1), lambda i: (0, 0)),
            pl.BlockSpec((128, 64), lambda i: (0, 0)),
            pl.BlockSpec((128, 1), lambda i: (0, 0)),
            pl.BlockSpec((32, 128), lambda i: (0, 0)),
            pl.BlockSpec((32, 1), lambda i: (0, 0)),
            pl.BlockSpec((1, 32), lambda i: (0, 0)),
            pl.BlockSpec((1, 1), lambda i: (0, 0)),
        ],
        out_specs=pl.BlockSpec((1, _TILE), lambda i: (0, i)),
        out_shape=jax.ShapeDtypeStruct((1, _N), jnp.float32),
    )(src_pts, W1, b1.reshape(64, 1), W2, b2.reshape(128, 1), W3,
      b3.reshape(32, 1), wl_w.reshape(1, 32), wl_b.reshape(1, 1))
    return out.reshape(_N)


def kernel(src_pts, tgt_pts, W1, b1, W2, b2, W3, b3, wl_w, wl_b):
    mean_scores = _mean_scores(src_pts, W1, b1, W2, b2, W3, b3, wl_w, wl_b)
    _, idx = jax.lax.top_k(mean_scores, _K)
    keypts = jnp.take(src_pts, idx, axis=2)
    return jnp.transpose(keypts, (0, 2, 1))
